# Initial kernel scaffold; baseline (speedup 1.0000x reference)
import jax
import jax.numpy as jnp
from jax.experimental import pallas as pl


def kernel(feature_hashes, feature_weights, weight):
    raise NotImplementedError("write your pallas kernel here")


# speedup vs baseline: 2.5202x; 2.5202x over previous
import functools

import jax
import jax.numpy as jnp
from jax import lax
from jax.experimental import pallas as pl
from jax.experimental.pallas import tpu as pltpu
from jax.experimental.pallas import tpu_sc as plsc

B = 16384
L = 50
D = 64
LANES = 16

NC = 2
NS = 16
NW = NC * NS

BPC = 16
ROWS = BPC * L
NDMA = 10
RPD = ROWS // NDMA
CHUNKS = B // BPC
CPW = CHUNKS // NW
DG = D // LANES
WPAD = 64


def _rsqrt_newton(sv):
    bits = plsc.bitcast(sv, jnp.int32)
    y = plsc.bitcast(jnp.full((LANES,), 0x5F3759DF, jnp.int32) - (bits >> 1),
                     jnp.float32)
    hv = sv * 0.5
    for _ in range(3):
        y = y * (1.5 - hv * y * y)
    return y


def _body(idx_hbm, w_hbm, table_hbm, out_hbm,
          idx_v, gath_v, w_v, outb_v, gsems):
    wid = lax.axis_index("c") * NS + lax.axis_index("s")

    def gather_copies(g, buf):
        return [
            pltpu.make_async_copy(
                table_hbm.at[idx_v.at[buf, j]],
                gath_v.at[buf, pl.ds(j * RPD, RPD)],
                gsems.at[buf],
            )
            for j in range(NDMA)
        ]

    def stage(g, buf):
        pltpu.sync_copy(idx_hbm.at[g], idx_v.at[buf])
        for cp in gather_copies(g, buf):
            cp.start()
        pltpu.sync_copy(w_hbm.at[pl.ds(g * BPC, BPC)], w_v.at[buf])

    def compute(g, buf):
        for cp in gather_copies(g, buf):
            cp.wait()

        def per_row(b, carry):
            acc = [jnp.zeros((LANES,), jnp.float32) for _ in range(DG)]
            base = b * L
            wvecs = [w_v[buf, b, pl.ds(k * LANES, LANES)]
                     for k in range(WPAD // LANES)]
            for l in range(L):
                w = wvecs[l // LANES][l % LANES]
                for d in range(DG):
                    acc[d] = acc[d] + w * gath_v[buf, base + l,
                                                 pl.ds(d * LANES, LANES)]
            for d in range(DG):
                outb_v[buf, b, pl.ds(d * LANES, LANES)] = acc[d]
            return carry

        lax.fori_loop(0, BPC, per_row, 0)

        bidx = lax.iota(jnp.int32, LANES)
        blk = outb_v.at[buf]
        ss = jnp.zeros((LANES,), jnp.float32)
        for d in range(D):
            didx = jnp.full((LANES,), d, jnp.int32)
            v = plsc.load_gather(blk, [bidx, didx])
            ss = ss + v * v
        inv = _rsqrt_newton(jnp.maximum(ss, 1e-24))
        for d in range(D):
            didx = jnp.full((LANES,), d, jnp.int32)
            v = plsc.load_gather(blk, [bidx, didx])
            plsc.store_scatter(blk, [bidx, didx], v * inv)
        pltpu.sync_copy(outb_v.at[buf], out_hbm.at[pl.ds(g * BPC, BPC)])

    g0 = wid * CPW
    stage(g0, 0)

    def chunk_iter(i, carry):
        for sub in range(2):
            c = i * 2 + sub
            g = g0 + c

            @pl.when(c + 1 < CPW)
            def _():
                stage(g + 1, 1 - sub)

            compute(g, sub)
        return carry

    lax.fori_loop(0, CPW // 2, chunk_iter, 0)


@functools.partial(
    pl.kernel,
    out_type=jax.ShapeDtypeStruct((B, D), jnp.float32),
    mesh=plsc.VectorSubcoreMesh(core_axis_name="c", subcore_axis_name="s"),
    compiler_params=pltpu.CompilerParams(needs_layout_passes=False,
                                         use_tc_tiling_on_sc=False),
    scratch_types=[
        pltpu.VMEM((2, NDMA, RPD), jnp.int32),
        pltpu.VMEM((2, ROWS, D), jnp.float32),
        pltpu.VMEM((2, BPC, WPAD), jnp.float32),
        pltpu.VMEM((2, BPC, D), jnp.float32),
        pltpu.SemaphoreType.DMA((2,)),
    ],
)
def _sc_embedding_bag(idx_hbm, w_hbm, table_hbm, out_hbm,
                      idx_v, gath_v, w_v, outb_v, gsems):
    _body(idx_hbm, w_hbm, table_hbm, out_hbm,
          idx_v, gath_v, w_v, outb_v, gsems)


def kernel(feature_hashes, feature_weights, weight):
    idx = feature_hashes.astype(jnp.int32).reshape(CHUNKS, NDMA, RPD)
    w = jnp.pad(feature_weights, ((0, 0), (0, WPAD - L)))
    return _sc_embedding_bag(idx, w, weight)


# speedup vs baseline: 2.6498x; 1.0514x over previous
import functools

import jax
import jax.numpy as jnp
from jax import lax
from jax.experimental import pallas as pl
from jax.experimental.pallas import tpu as pltpu
from jax.experimental.pallas import tpu_sc as plsc

B = 16384
L = 50
D = 64
LANES = 16

NC = 2
NS = 16
NW = NC * NS

BPC = 16
ROWS = BPC * L
NDMA = 10
RPD = ROWS // NDMA
CHUNKS = B // BPC
CPW = CHUNKS // NW
DG = D // LANES
WPAD = 64

_GDN = lax.GatherDimensionNumbers(
    offset_dims=(), collapsed_slice_dims=(0,), start_index_map=(0,))


def _lane_bcast(v, l):
    idx = jnp.full((LANES, 1), l, jnp.int32)
    return lax.gather(v, idx, _GDN, slice_sizes=(1,),
                      mode=lax.GatherScatterMode.PROMISE_IN_BOUNDS)


def _rsqrt_newton(sv):
    bits = plsc.bitcast(sv, jnp.int32)
    y = plsc.bitcast(jnp.full((LANES,), 0x5F3759DF, jnp.int32) - (bits >> 1),
                     jnp.float32)
    hv = sv * 0.5
    for _ in range(3):
        y = y * (1.5 - hv * y * y)
    return y


def _body(idx_hbm, w_hbm, table_hbm, out_hbm,
          idx_v, gath_v, w_v, outb_v, gsems, isems, wsems, osems):
    wid = lax.axis_index("c") * NS + lax.axis_index("s")
    g0 = wid * CPW

    def gather_copies(buf):
        return [
            pltpu.make_async_copy(
                table_hbm.at[idx_v.at[buf, j]],
                gath_v.at[buf, pl.ds(j * RPD, RPD)],
                gsems.at[buf],
            )
            for j in range(NDMA)
        ]

    def idx_copy(g, buf):
        return pltpu.make_async_copy(idx_hbm.at[g], idx_v.at[buf],
                                     isems.at[buf])

    def w_copy(g, buf):
        return pltpu.make_async_copy(w_hbm.at[pl.ds(g * BPC, BPC)],
                                     w_v.at[buf], wsems.at[buf])

    def out_copy(g, buf):
        return pltpu.make_async_copy(outb_v.at[buf],
                                     out_hbm.at[pl.ds(g * BPC, BPC)],
                                     osems.at[buf])

    def compute(buf):
        def per_row(b, carry):
            base = b * L
            wvecs = [w_v[buf, b, pl.ds(k * LANES, LANES)]
                     for k in range(WPAD // LANES)]
            acc = [jnp.zeros((LANES,), jnp.float32) for _ in range(2 * DG)]
            for l in range(L):
                w = _lane_bcast(wvecs[l // LANES], l % LANES)
                for d in range(DG):
                    a = (l % 2) * DG + d
                    acc[a] = acc[a] + w * gath_v[buf, base + l,
                                                 pl.ds(d * LANES, LANES)]
            for d in range(DG):
                outb_v[buf, b, pl.ds(d * LANES, LANES)] = acc[d] + acc[DG + d]
            return carry

        lax.fori_loop(0, BPC, per_row, 0)

        bidx = lax.iota(jnp.int32, LANES)
        blk = outb_v.at[buf]
        ss = jnp.zeros((LANES,), jnp.float32)
        for d in range(D):
            didx = jnp.full((LANES,), d, jnp.int32)
            v = plsc.load_gather(blk, [bidx, didx])
            ss = ss + v * v
        inv = _rsqrt_newton(jnp.maximum(ss, 1e-24))
        for d in range(D):
            didx = jnp.full((LANES,), d, jnp.int32)
            v = plsc.load_gather(blk, [bidx, didx])
            plsc.store_scatter(blk, [bidx, didx], v * inv)

    idx_copy(g0, 0).start()
    w_copy(g0, 0).start()
    idx_copy(g0 + 1, 1).start()
    w_copy(g0 + 1, 1).start()
    idx_copy(g0, 0).wait()
    for cp in gather_copies(0):
        cp.start()

    def chunk_iter(i, carry):
        for sub in range(2):
            c = i * 2 + sub
            g = g0 + c

            for cp in gather_copies(sub):
                cp.wait()

            @pl.when(c + 1 < CPW)
            def _():
                idx_copy(g + 1, 1 - sub).wait()
                for cp in gather_copies(1 - sub):
                    cp.start()

            @pl.when(c + 2 < CPW)
            def _():
                idx_copy(g + 2, sub).start()

            w_copy(g, sub).wait()

            @pl.when(c >= 2)
            def _():
                out_copy(g - 2, sub).wait()

            compute(sub)
            out_copy(g, sub).start()

            @pl.when(c + 2 < CPW)
            def _():
                w_copy(g + 2, sub).start()
        return carry

    lax.fori_loop(0, CPW // 2, chunk_iter, 0)
    out_copy(g0 + CPW - 2, 0).wait()
    out_copy(g0 + CPW - 1, 1).wait()


@functools.partial(
    pl.kernel,
    out_type=jax.ShapeDtypeStruct((B, D), jnp.float32),
    mesh=plsc.VectorSubcoreMesh(core_axis_name="c", subcore_axis_name="s"),
    compiler_params=pltpu.CompilerParams(needs_layout_passes=False,
                                         use_tc_tiling_on_sc=False),
    scratch_types=[
        pltpu.VMEM((2, NDMA, RPD), jnp.int32),
        pltpu.VMEM((2, ROWS, D), jnp.float32),
        pltpu.VMEM((2, BPC, WPAD), jnp.float32),
        pltpu.VMEM((2, BPC, D), jnp.float32),
        pltpu.SemaphoreType.DMA((2,)),
        pltpu.SemaphoreType.DMA((2,)),
        pltpu.SemaphoreType.DMA((2,)),
        pltpu.SemaphoreType.DMA((2,)),
    ],
)
def _sc_embedding_bag(idx_hbm, w_hbm, table_hbm, out_hbm,
                      idx_v, gath_v, w_v, outb_v, gsems, isems, wsems, osems):
    _body(idx_hbm, w_hbm, table_hbm, out_hbm,
          idx_v, gath_v, w_v, outb_v, gsems, isems, wsems, osems)


def kernel(feature_hashes, feature_weights, weight):
    idx = feature_hashes.astype(jnp.int32).reshape(CHUNKS, NDMA, RPD)
    w = jnp.pad(feature_weights, ((0, 0), (0, WPAD - L)))
    table = lax.optimization_barrier(weight)
    return _sc_embedding_bag(idx, w, table)
